# Initial kernel scaffold; baseline (speedup 1.0000x reference)
import jax
import jax.numpy as jnp
from jax.experimental import pallas as pl


def kernel(output_in, output_out, interface_mask):
    raise NotImplementedError("write your pallas kernel here")


# speedup vs baseline: 893.2668x; 893.2668x over previous
import jax
import jax.numpy as jnp
from jax.experimental import pallas as pl
from jax.experimental.pallas import tpu as pltpu

WEIGHT = 10.0
EPS1 = 1.0
EPS2 = 80.0
DX = 0.001953125
DY = 0.001953125
CENTER = (256.5, 256.5)
B, H, W = 16, 512, 512


def _loss_kernel(phi1_ref, phi2_ref, mask_ref, out_ref):
    b = pl.program_id(0)
    phi1 = phi1_ref[0]
    phi2 = phi2_ref[0]
    m = mask_ref[...]

    inv2dx = 1.0 / (2.0 * DX)
    inv2dy = 1.0 / (2.0 * DY)

    def ddx(phi):
        d = (phi[:, 2:] - phi[:, :-2]) * inv2dx
        return jnp.concatenate([d[:, :1], d, d[:, -1:]], axis=1)

    def ddy(phi):
        d = (phi[2:, :] - phi[:-2, :]) * inv2dy
        return jnp.concatenate([d[:1, :], d, d[-1:, :]], axis=0)

    cx = jax.lax.broadcasted_iota(jnp.int32, (H, W), 1).astype(jnp.float32) - CENTER[0]
    cy = jax.lax.broadcasted_iota(jnp.int32, (H, W), 0).astype(jnp.float32) - CENTER[1]
    inv_norm = 1.0 / jnp.sqrt(cx * cx + cy * cy)
    nxf = cx * inv_norm
    nyf = cy * inv_norm

    dn1 = nxf * ddx(phi1) + nyf * ddy(phi1)
    dn2 = nxf * ddx(phi2) + nyf * ddy(phi2)
    mismatch = EPS1 * dn1 - EPS2 * dn2

    diff = phi1 - phi2
    t = (diff * diff + mismatch * mismatch) * m
    partial = jnp.sum(t)

    @pl.when(b == 0)
    def _():
        out_ref[0, 0] = 0.0

    out_ref[0, 0] += partial

    @pl.when(b == B - 1)
    def _():
        count = jnp.sum(m)
        out_ref[0, 0] = out_ref[0, 0] * (WEIGHT / B) / count


def kernel(output_in, output_out, interface_mask):
    phi1 = output_in[:, 0]
    phi2 = output_out[:, 0]
    maskf = interface_mask.astype(jnp.float32)

    out = pl.pallas_call(
        _loss_kernel,
        grid=(B,),
        in_specs=[
            pl.BlockSpec((1, H, W), lambda b: (b, 0, 0)),
            pl.BlockSpec((1, H, W), lambda b: (b, 0, 0)),
            pl.BlockSpec((H, W), lambda b: (0, 0)),
        ],
        out_specs=pl.BlockSpec(memory_space=pltpu.SMEM),
        out_shape=jax.ShapeDtypeStruct((1, 1), jnp.float32),
    )(phi1, phi2, maskf)
    return out[0, 0]


# speedup vs baseline: 988.9029x; 1.1071x over previous
import jax
import jax.numpy as jnp
from jax.experimental import pallas as pl
from jax.experimental.pallas import tpu as pltpu

WEIGHT = 10.0
EPS1 = 1.0
EPS2 = 80.0
DX = 0.001953125
DY = 0.001953125
CENTER = (256.5, 256.5)
B, H, W = 16, 512, 512


def _loss_kernel(phi1_ref, phi2_ref, mask_ref, out_ref, cx_ref, cy_ref, irr_ref):
    b = pl.program_id(0)

    @pl.when(b == 0)
    def _():
        cx = jax.lax.broadcasted_iota(jnp.int32, (H, W), 1).astype(jnp.float32) - CENTER[0]
        cy = jax.lax.broadcasted_iota(jnp.int32, (H, W), 0).astype(jnp.float32) - CENTER[1]
        inv2d = 1.0 / (2.0 * DX)
        cx_ref[...] = cx
        cy_ref[...] = cy
        irr_ref[...] = (inv2d * inv2d) / (cx * cx + cy * cy)
        out_ref[0, 0] = 0.0

    phi1 = phi1_ref[0]
    phi2 = phi2_ref[0]
    m = mask_ref[...]

    def ddx(phi):
        d = phi[:, 2:] - phi[:, :-2]
        return jnp.concatenate([d[:, :1], d, d[:, -1:]], axis=1)

    def ddy(phi):
        d = phi[2:, :] - phi[:-2, :]
        return jnp.concatenate([d[:1, :], d, d[-1:, :]], axis=0)

    ax = ddx(phi1) - EPS2 * ddx(phi2)
    ay = ddy(phi1) - EPS2 * ddy(phi2)
    u = cx_ref[...] * ax + cy_ref[...] * ay
    diff = phi1 - phi2
    t = (diff * diff + u * u * irr_ref[...]) * m
    out_ref[0, 0] += jnp.sum(t)

    @pl.when(b == B - 1)
    def _():
        count = jnp.sum(m)
        out_ref[0, 0] = out_ref[0, 0] * (WEIGHT / B) / count


def kernel(output_in, output_out, interface_mask):
    phi1 = output_in[:, 0]
    phi2 = output_out[:, 0]
    maskf = interface_mask.astype(jnp.float32)

    out = pl.pallas_call(
        _loss_kernel,
        grid=(B,),
        in_specs=[
            pl.BlockSpec((1, H, W), lambda b: (b, 0, 0)),
            pl.BlockSpec((1, H, W), lambda b: (b, 0, 0)),
            pl.BlockSpec((H, W), lambda b: (0, 0)),
        ],
        out_specs=pl.BlockSpec(memory_space=pltpu.SMEM),
        out_shape=jax.ShapeDtypeStruct((1, 1), jnp.float32),
        scratch_shapes=[
            pltpu.VMEM((H, W), jnp.float32),
            pltpu.VMEM((H, W), jnp.float32),
            pltpu.VMEM((H, W), jnp.float32),
        ],
    )(phi1, phi2, maskf)
    return out[0, 0]


# speedup vs baseline: 1238.7308x; 1.2526x over previous
import jax
import jax.numpy as jnp
from jax.experimental import pallas as pl
from jax.experimental.pallas import tpu as pltpu

WEIGHT = 10.0
EPS1 = 1.0
EPS2 = 80.0
DX = 0.001953125
DY = 0.001953125
CENTER = (256.5, 256.5)
B, H, W = 16, 512, 512


def _loss_kernel(phi1_ref, phi2_ref, mask_ref, out_ref, cx_ref, cy_ref, irr_ref):
    b = pl.program_id(0)

    @pl.when(b == 0)
    def _():
        cx = jax.lax.broadcasted_iota(jnp.int32, (H, W), 1).astype(jnp.float32) - CENTER[0]
        cy = jax.lax.broadcasted_iota(jnp.int32, (H, W), 0).astype(jnp.float32) - CENTER[1]
        inv2d = 1.0 / (2.0 * DX)
        cx_ref[...] = cx
        cy_ref[...] = cy
        irr_ref[...] = (inv2d * inv2d) / (cx * cx + cy * cy)
        out_ref[0, 0] = 0.0

    phi1 = phi1_ref[0]
    phi2 = phi2_ref[0]
    m = mask_ref[...]

    def ddx(phi):
        d = phi[:, 2:] - phi[:, :-2]
        return jnp.concatenate([d[:, :1], d, d[:, -1:]], axis=1)

    def ddy(phi):
        d = phi[2:, :] - phi[:-2, :]
        return jnp.concatenate([d[:1, :], d, d[-1:, :]], axis=0)

    q = phi1 - EPS2 * phi2
    u = cx_ref[...] * ddx(q) + cy_ref[...] * ddy(q)
    diff = phi1 - phi2
    t = (diff * diff + u * u * irr_ref[...]) * m
    out_ref[0, 0] += jnp.sum(t)

    @pl.when(b == B - 1)
    def _():
        count = jnp.sum(m)
        out_ref[0, 0] = out_ref[0, 0] * (WEIGHT / B) / count


def kernel(output_in, output_out, interface_mask):
    phi1 = output_in[:, 0]
    phi2 = output_out[:, 0]
    maskf = interface_mask.astype(jnp.float32)

    out = pl.pallas_call(
        _loss_kernel,
        grid=(B,),
        in_specs=[
            pl.BlockSpec((1, H, W), lambda b: (b, 0, 0)),
            pl.BlockSpec((1, H, W), lambda b: (b, 0, 0)),
            pl.BlockSpec((H, W), lambda b: (0, 0)),
        ],
        out_specs=pl.BlockSpec(memory_space=pltpu.SMEM),
        out_shape=jax.ShapeDtypeStruct((1, 1), jnp.float32),
        scratch_shapes=[
            pltpu.VMEM((H, W), jnp.float32),
            pltpu.VMEM((H, W), jnp.float32),
            pltpu.VMEM((H, W), jnp.float32),
        ],
    )(phi1, phi2, maskf)
    return out[0, 0]
